# Initial kernel scaffold; baseline (speedup 1.0000x reference)
import jax
import jax.numpy as jnp
from jax.experimental import pallas as pl


def kernel(lookup, table):
    raise NotImplementedError("write your pallas kernel here")


# speedup vs baseline: 2.4873x; 2.4873x over previous
import functools

import jax
import jax.numpy as jnp
from jax import lax
from jax.experimental import pallas as pl
from jax.experimental.pallas import tpu as pltpu
from jax.experimental.pallas import tpu_sc as plsc

BATCH = 16384
HIST = 200
D_FIELD = 16
B_FLAT = BATCH * HIST


@functools.cache
def _build(n_rows, n_vocab):
    info = plsc.get_sparse_core_info()
    nw = info.num_cores * info.num_subcores
    b_per_w = n_rows // nw
    chunk = 2048
    n_chunks = b_per_w // chunk

    mesh = plsc.VectorSubcoreMesh(core_axis_name="c", subcore_axis_name="s")

    @functools.partial(
        pl.kernel,
        mesh=mesh,
        out_type=jax.ShapeDtypeStruct((n_rows, D_FIELD), jnp.float32),
        scratch_types=[
            pltpu.VMEM((chunk,), jnp.int32),
            pltpu.VMEM((chunk, D_FIELD), jnp.float32),
            pltpu.SemaphoreType.DMA,
        ],
        compiler_params=pltpu.CompilerParams(use_tc_tiling_on_sc=False),
    )
    def gather_kernel(idx_hbm, table_hbm, out_hbm, idx_v, rows_v, sem):
        wid = lax.axis_index("s") * info.num_cores + lax.axis_index("c")
        base = wid * b_per_w

        def body(i, carry):
            off = base + i * chunk
            pltpu.sync_copy(idx_hbm.at[pl.ds(off, chunk)], idx_v)
            pltpu.async_copy(table_hbm.at[idx_v], rows_v, sem).wait()
            pltpu.sync_copy(rows_v, out_hbm.at[pl.ds(off, chunk)])
            return carry

        lax.fori_loop(0, n_chunks, body, 0)

    return gather_kernel


def kernel(lookup, table):
    idx = lookup.reshape(B_FLAT).astype(jnp.int32)
    out = _build(B_FLAT, table.shape[0])(idx, table)
    return out.reshape(BATCH, HIST, D_FIELD)


# speedup vs baseline: 2.5445x; 1.0230x over previous
import functools

import jax
import jax.numpy as jnp
from jax import lax
from jax.experimental import pallas as pl
from jax.experimental.pallas import tpu as pltpu
from jax.experimental.pallas import tpu_sc as plsc

BATCH = 16384
HIST = 200
D_FIELD = 16
B_FLAT = BATCH * HIST

CHUNK = 3200
NBUF = 2


@functools.cache
def _build(n_rows, n_vocab):
    info = plsc.get_sparse_core_info()
    nw = info.num_cores * info.num_subcores
    b_per_w = n_rows // nw
    n_chunks = b_per_w // CHUNK
    n_pairs = n_chunks // NBUF

    mesh = plsc.VectorSubcoreMesh(core_axis_name="c", subcore_axis_name="s")

    @functools.partial(
        pl.kernel,
        mesh=mesh,
        out_type=jax.ShapeDtypeStruct((n_rows, D_FIELD), jnp.float32),
        scratch_types=[
            pltpu.VMEM((NBUF, CHUNK), jnp.int32),
            pltpu.VMEM((NBUF, CHUNK, D_FIELD), jnp.float32),
            pltpu.SemaphoreType.DMA((NBUF,)),
            pltpu.SemaphoreType.DMA((NBUF,)),
            pltpu.SemaphoreType.DMA((NBUF,)),
        ],
        compiler_params=pltpu.CompilerParams(use_tc_tiling_on_sc=False),
    )
    def gather_kernel(idx_hbm, table_hbm, out_hbm, idx_v, rows_v, sem_i, sem_g, sem_o):
        wid = lax.axis_index("s") * info.num_cores + lax.axis_index("c")
        base = wid * b_per_w

        for b in range(NBUF):
            pltpu.async_copy(
                idx_hbm.at[pl.ds(base + b * CHUNK, CHUNK)], idx_v.at[b], sem_i.at[b]
            )

        def pair_body(p, carry):
            for b in range(NBUF):
                i = p * NBUF + b
                off = base + i * CHUNK

                @pl.when(p >= 1)
                def _wait_out():
                    pltpu.make_async_copy(
                        rows_v.at[b], out_hbm.at[pl.ds(off, CHUNK)], sem_o.at[b]
                    ).wait()

                pltpu.make_async_copy(
                    idx_hbm.at[pl.ds(off, CHUNK)], idx_v.at[b], sem_i.at[b]
                ).wait()

                pltpu.async_copy(
                    table_hbm.at[idx_v.at[b]], rows_v.at[b], sem_g.at[b]
                ).wait()

                pltpu.async_copy(
                    rows_v.at[b], out_hbm.at[pl.ds(off, CHUNK)], sem_o.at[b]
                )

                @pl.when(p + 1 < n_pairs)
                def _prefetch():
                    pltpu.async_copy(
                        idx_hbm.at[pl.ds(off + NBUF * CHUNK, CHUNK)],
                        idx_v.at[b],
                        sem_i.at[b],
                    )

            return carry

        lax.fori_loop(0, n_pairs, pair_body, 0)

        for b in range(NBUF):
            pltpu.make_async_copy(
                rows_v.at[b], out_hbm.at[pl.ds(base, CHUNK)], sem_o.at[b]
            ).wait()

    return gather_kernel


def kernel(lookup, table):
    idx = lookup.reshape(B_FLAT).astype(jnp.int32)
    out = _build(B_FLAT, table.shape[0])(idx, table)
    return out.reshape(BATCH, HIST, D_FIELD)


# speedup vs baseline: 2.5451x; 1.0002x over previous
import functools

import jax
import jax.numpy as jnp
from jax import lax
from jax.experimental import pallas as pl
from jax.experimental.pallas import tpu as pltpu
from jax.experimental.pallas import tpu_sc as plsc

BATCH = 16384
HIST = 200
D_FIELD = 16
B_FLAT = BATCH * HIST

CHUNK = 3200
NBUF = 2
K_SUB = 8
S_SUB = CHUNK // K_SUB


@functools.cache
def _build(n_rows, n_vocab):
    info = plsc.get_sparse_core_info()
    nw = info.num_cores * info.num_subcores
    b_per_w = n_rows // nw
    n_chunks = b_per_w // CHUNK
    n_pairs = n_chunks // NBUF

    mesh = plsc.VectorSubcoreMesh(core_axis_name="c", subcore_axis_name="s")

    @functools.partial(
        pl.kernel,
        mesh=mesh,
        out_type=jax.ShapeDtypeStruct((n_rows, D_FIELD), jnp.float32),
        scratch_types=[
            pltpu.VMEM((NBUF, CHUNK), jnp.int32),
            pltpu.VMEM((NBUF, CHUNK, D_FIELD), jnp.float32),
            pltpu.SemaphoreType.DMA((NBUF,)),
            pltpu.SemaphoreType.DMA((NBUF,)),
            pltpu.SemaphoreType.DMA((NBUF,)),
        ],
        compiler_params=pltpu.CompilerParams(use_tc_tiling_on_sc=False),
    )
    def gather_kernel(idx_hbm, table_hbm, out_hbm, idx_v, rows_v, sem_i, sem_g, sem_o):
        wid = lax.axis_index("s") * info.num_cores + lax.axis_index("c")
        base = wid * b_per_w

        for b in range(NBUF):
            pltpu.async_copy(
                idx_hbm.at[pl.ds(base + b * CHUNK, CHUNK)], idx_v.at[b], sem_i.at[b]
            )

        def pair_body(p, carry):
            for b in range(NBUF):
                i = p * NBUF + b
                off = base + i * CHUNK

                @pl.when(p >= 1)
                def _wait_out():
                    pltpu.make_async_copy(
                        rows_v.at[b], out_hbm.at[pl.ds(off, CHUNK)], sem_o.at[b]
                    ).wait()

                pltpu.make_async_copy(
                    idx_hbm.at[pl.ds(off, CHUNK)], idx_v.at[b], sem_i.at[b]
                ).wait()

                for j in range(K_SUB):
                    pltpu.async_copy(
                        table_hbm.at[idx_v.at[b, pl.ds(j * S_SUB, S_SUB)]],
                        rows_v.at[b, pl.ds(j * S_SUB, S_SUB)],
                        sem_g.at[b],
                    )
                for j in range(K_SUB):
                    pltpu.make_async_copy(
                        table_hbm.at[idx_v.at[b, pl.ds(j * S_SUB, S_SUB)]],
                        rows_v.at[b, pl.ds(j * S_SUB, S_SUB)],
                        sem_g.at[b],
                    ).wait()

                pltpu.async_copy(
                    rows_v.at[b], out_hbm.at[pl.ds(off, CHUNK)], sem_o.at[b]
                )

                @pl.when(p + 1 < n_pairs)
                def _prefetch():
                    pltpu.async_copy(
                        idx_hbm.at[pl.ds(off + NBUF * CHUNK, CHUNK)],
                        idx_v.at[b],
                        sem_i.at[b],
                    )

            return carry

        lax.fori_loop(0, n_pairs, pair_body, 0)

        for b in range(NBUF):
            pltpu.make_async_copy(
                rows_v.at[b], out_hbm.at[pl.ds(base, CHUNK)], sem_o.at[b]
            ).wait()

    return gather_kernel


def kernel(lookup, table):
    idx = lookup.reshape(B_FLAT).astype(jnp.int32)
    out = _build(B_FLAT, table.shape[0])(idx, table)
    return out.reshape(BATCH, HIST, D_FIELD)
